# baseline (device time: 39961 ns/iter reference)
import os

import jax
import jax.numpy as jnp
from jax import lax
from jax.experimental import pallas as pl
from jax.experimental.pallas import tpu as pltpu

N_DEV = 4
_NO_COMM = os.environ.get("KERNEL_NO_COMM") == "1"


def kernel(x, Win0, Wout0, Win1, Wout1, Win2, Wout2):
    M, D = x.shape
    H = Win0.shape[1]
    B = N_DEV * M

    def body(xh_ref, win0h_ref, wout0h_ref, win1h_ref, wout1h_ref,
             win2h_ref, wout2h_ref, out_ref, xin, winb, woutb, X0, X1, X2,
             prbuf, sbuf, load_sems, send_sems, recv_sems):
        my = lax.axis_index("i")

        x_load = pltpu.make_async_copy(xh_ref, xin, load_sems.at[0])
        x_load.start()
        w_loads = []
        for l, (winh, wouth) in enumerate(((win0h_ref, wout0h_ref),
                                           (win1h_ref, wout1h_ref),
                                           (win2h_ref, wout2h_ref))):
            ldi = pltpu.make_async_copy(winh, winb.at[l], load_sems.at[1 + 2 * l])
            ldo = pltpu.make_async_copy(wouth, woutb.at[l], load_sems.at[2 + 2 * l])
            ldi.start()
            ldo.start()
            w_loads.append((ldi, ldo))

        if not _NO_COMM:
            barrier_sem = pltpu.get_barrier_semaphore()
            for k in range(1, N_DEV):
                pl.semaphore_signal(
                    barrier_sem, inc=1,
                    device_id=((my + k) % N_DEV,),
                    device_id_type=pl.DeviceIdType.MESH,
                )
            pl.semaphore_wait(barrier_sem, N_DEV - 1)

        def compute_chunk(xc, l):
            w_in = winb[l, :, :].astype(jnp.bfloat16)
            h = jnp.dot(xc, w_in, preferred_element_type=jnp.float32)
            h = jnp.maximum(h, 0.0).astype(jnp.bfloat16)
            w_out = woutb[l, :, :].astype(jnp.bfloat16)
            return jnp.dot(h, w_out, preferred_element_type=jnp.float32)

        def layer_step(l, Xl, x_own):
            if _NO_COMM:
                total = compute_chunk(x_own, l)
                for _ in range(3):
                    total = total + compute_chunk(x_own, l)
                return total
            Xl[pl.ds(my * M, M), :] = x_own
            xdescs = []
            for k in range(1, N_DEV):
                t = (my + k) % N_DEV
                de = pltpu.make_async_remote_copy(
                    src_ref=Xl.at[pl.ds(my * M, M), :],
                    dst_ref=Xl.at[pl.ds(my * M, M), :],
                    send_sem=send_sems.at[2 * l, k - 1],
                    recv_sem=recv_sems.at[2 * l, k - 1],
                    device_id=(t,),
                    device_id_type=pl.DeviceIdType.MESH,
                )
                de.start()
                xdescs.append(de)

            total = compute_chunk(x_own, l)

            pdescs = []
            for k in (1, 3, 2):
                xdescs[k - 1].wait_recv()
                s = (my - k) % N_DEV
                xk = Xl[pl.ds(s * M, M), :]
                pk = compute_chunk(xk, l)
                slot = 3 - k
                sbuf[l, slot, :, :] = pk.astype(jnp.bfloat16)
                de = pltpu.make_async_remote_copy(
                    src_ref=sbuf.at[l, slot],
                    dst_ref=prbuf.at[l, slot],
                    send_sem=send_sems.at[2 * l + 1, slot],
                    recv_sem=recv_sems.at[2 * l + 1, slot],
                    device_id=(s,),
                    device_id_type=pl.DeviceIdType.MESH,
                )
                de.start()
                pdescs.append(de)

            for de, k in zip(pdescs, (1, 3, 2)):
                de.wait_recv()
                total = total + prbuf[l, 3 - k].astype(jnp.float32)
            for de in xdescs + pdescs:
                de.wait_send()
            return total

        x_load.wait()
        for ldi, ldo in w_loads[:1]:
            ldi.wait()
            ldo.wait()
        x0 = xin[:, :].astype(jnp.bfloat16)
        r0 = layer_step(0, X0, x0)
        for ldi, ldo in w_loads[1:2]:
            ldi.wait()
            ldo.wait()
        r1 = layer_step(1, X1, r0.astype(jnp.bfloat16))
        for ldi, ldo in w_loads[2:]:
            ldi.wait()
            ldo.wait()
        r2 = layer_step(2, X2, r1.astype(jnp.bfloat16))
        out_ref[:, :] = r2

    return pl.pallas_call(
        body,
        out_shape=jax.ShapeDtypeStruct((M, D), jnp.float32),
        in_specs=[pl.BlockSpec(memory_space=pl.ANY)] * 7,
        out_specs=pl.BlockSpec(memory_space=pltpu.VMEM),
        scratch_shapes=[
            pltpu.VMEM((M, D), jnp.float32),
            pltpu.VMEM((3, D, H), jnp.float32),
            pltpu.VMEM((3, H, D), jnp.float32),
            pltpu.VMEM((B, D), jnp.bfloat16),
            pltpu.VMEM((B, D), jnp.bfloat16),
            pltpu.VMEM((B, D), jnp.bfloat16),
            pltpu.VMEM((3, N_DEV - 1, M, D), jnp.bfloat16),
            pltpu.VMEM((3, N_DEV - 1, M, D), jnp.bfloat16),
            pltpu.SemaphoreType.DMA((7,)),
            pltpu.SemaphoreType.DMA((6, N_DEV - 1)),
            pltpu.SemaphoreType.DMA((6, N_DEV - 1)),
        ],
        compiler_params=pltpu.CompilerParams(
            collective_id=None if _NO_COMM else 0
        ),
    )(x, Win0, Wout0, Win1, Wout1, Win2, Wout2)


# device time: 37197 ns/iter; 1.0743x vs baseline; 1.0743x over previous
import jax
import jax.numpy as jnp
from jax import lax
from jax.experimental import pallas as pl
from jax.experimental.pallas import tpu as pltpu

N_DEV = 4


def kernel(x, Win0, Wout0, Win1, Wout1, Win2, Wout2):
    M, D = x.shape
    B = N_DEV * M
    M2 = M // 2

    def body(x_ref, win0_ref, wout0_ref, win1_ref, wout1_ref, win2_ref,
             wout2_ref, out_ref, X0, X1, X2, prbuf, sbuf,
             send_sems, recv_sems):
        my = lax.axis_index("i")
        weights = ((win0_ref, wout0_ref), (win1_ref, wout1_ref),
                   (win2_ref, wout2_ref))
        all_descs = []

        barrier_sem = pltpu.get_barrier_semaphore()
        for k in range(1, N_DEV):
            pl.semaphore_signal(
                barrier_sem, inc=1,
                device_id=((my + k) % N_DEV,),
                device_id_type=pl.DeviceIdType.MESH,
            )
        pl.semaphore_wait(barrier_sem, N_DEV - 1)

        def compute_half(xc, l):
            win_ref, wout_ref = weights[l]
            w_in = win_ref[:, :].astype(jnp.bfloat16)
            h = jnp.dot(xc, w_in, preferred_element_type=jnp.float32)
            h = jnp.maximum(h, 0.0).astype(jnp.bfloat16)
            w_out = wout_ref[:, :].astype(jnp.bfloat16)
            return jnp.dot(h, w_out, preferred_element_type=jnp.float32)

        def layer_step(l, Xl, get_half):
            xdescs = {}
            own_p = {}
            for half in (0, 1):
                xh = get_half(half).astype(jnp.bfloat16)
                rows = pl.ds(my * M + half * M2, M2)
                Xl[rows, :] = xh
                for k in range(1, N_DEV):
                    t = (my + k) % N_DEV
                    sl = (k - 1) * 2 + half
                    de = pltpu.make_async_remote_copy(
                        src_ref=Xl.at[rows, :],
                        dst_ref=Xl.at[rows, :],
                        send_sem=send_sems.at[2 * l, sl],
                        recv_sem=recv_sems.at[2 * l, sl],
                        device_id=(t,),
                        device_id_type=pl.DeviceIdType.MESH,
                    )
                    de.start()
                    xdescs[(k, half)] = de
                own_p[half] = compute_half(xh, l)
            all_descs.extend(xdescs.values())

            pdescs = {}
            for k, half in ((1, 0), (3, 0), (1, 1), (3, 1), (2, 0), (2, 1)):
                xdescs[(k, half)].wait_recv()
                s = (my - k) % N_DEV
                xk = Xl[pl.ds(s * M + half * M2, M2), :]
                pk = compute_half(xk, l)
                psl = (3 - k) * 2 + half
                sbuf[l, psl, :, :] = pk.astype(jnp.bfloat16)
                de = pltpu.make_async_remote_copy(
                    src_ref=sbuf.at[l, psl],
                    dst_ref=prbuf.at[l, psl],
                    send_sem=send_sems.at[2 * l + 1, psl],
                    recv_sem=recv_sems.at[2 * l + 1, psl],
                    device_id=(s,),
                    device_id_type=pl.DeviceIdType.MESH,
                )
                de.start()
                pdescs[(k, half)] = de
            all_descs.extend(pdescs.values())

            def next_get_half(half):
                tot = own_p[half]
                for k in (1, 3, 2):
                    pdescs[(k, half)].wait_recv()
                    tot = tot + prbuf[l, (3 - k) * 2 + half].astype(
                        jnp.float32
                    )
                return tot

            return next_get_half

        get_half = lambda half: x_ref[pl.ds(half * M2, M2), :]
        get_half = layer_step(0, X0, get_half)
        get_half = layer_step(1, X1, get_half)
        get_half = layer_step(2, X2, get_half)
        for half in (0, 1):
            out_ref[pl.ds(half * M2, M2), :] = get_half(half)
        for de in all_descs:
            de.wait_send()

    return pl.pallas_call(
        body,
        out_shape=jax.ShapeDtypeStruct((M, D), jnp.float32),
        in_specs=[pl.BlockSpec(memory_space=pltpu.VMEM)] * 7,
        out_specs=pl.BlockSpec(memory_space=pltpu.VMEM),
        scratch_shapes=[
            pltpu.VMEM((B, D), jnp.bfloat16),
            pltpu.VMEM((B, D), jnp.bfloat16),
            pltpu.VMEM((B, D), jnp.bfloat16),
            pltpu.VMEM((3, 6, M2, D), jnp.bfloat16),
            pltpu.VMEM((3, 6, M2, D), jnp.bfloat16),
            pltpu.SemaphoreType.DMA((6, 6)),
            pltpu.SemaphoreType.DMA((6, 6)),
        ],
        compiler_params=pltpu.CompilerParams(collective_id=0),
    )(x, Win0, Wout0, Win1, Wout1, Win2, Wout2)
